# baseline (speedup 1.0000x reference)
import functools

import jax
import jax.numpy as jnp
from jax import lax
from jax.experimental import pallas as pl
from jax.experimental.pallas import tpu as pltpu
from jax.experimental.pallas import tpu_sc as plsc

_N = 10000
_M = 5000
_NNZ = 320000
_K = 128
_KE = 144
_MP = 5120
_NP = 10240
_NC = 2
_NS = 16
_NW = _NC * _NS
_PER_W = _NNZ // _NW
_EPS = 1e-12


def _chunks(total, step):
    out, o = [], 0
    while o < total:
        l = min(step, total - o)
        out.append((o, l))
        o += l
    return out


def _softmax_ext_body(y_ref, o_ref):
    y = y_ref[...]
    m = jnp.max(y, axis=1, keepdims=True)
    e = jnp.exp(y - m)
    p = e / jnp.sum(e, axis=1, keepdims=True)
    lane = lax.broadcasted_iota(jnp.int32, (y.shape[0], _KE - _K), 1)
    ext = jnp.where(lane == 0, 1.0, 0.0).astype(jnp.float32)
    o_ref[...] = jnp.concatenate([p, ext], axis=1)


def _softmax_ext(y):
    br = 1000
    return pl.pallas_call(
        _softmax_ext_body,
        grid=(_N // br,),
        in_specs=[pl.BlockSpec((br, _K), lambda i: (i, 0))],
        out_specs=pl.BlockSpec((br, _KE), lambda i: (i, 0)),
        out_shape=jax.ShapeDtypeStruct((_N, _KE), jnp.float32),
    )(y)


def _make_sc_segment_sum_stream_idx(S, G):
    rs = S // _NS
    C = _PER_W // G
    mesh = plsc.VectorSubcoreMesh(core_axis_name="c", subcore_axis_name="s")

    @functools.partial(
        pl.kernel,
        out_type=jax.ShapeDtypeStruct((_NC, S, _KE), jnp.float32),
        mesh=mesh,
        scratch_types=[
            pltpu.VMEM((G,), jnp.int32),
            pltpu.VMEM((G,), jnp.int32),
            pltpu.VMEM((G,), jnp.int32),
            pltpu.VMEM((G,), jnp.int32),
            pltpu.VMEM((G, _KE), jnp.float32),
            pltpu.VMEM((G, _KE), jnp.float32),
            pltpu.VMEM_SHARED((S, _KE), jnp.float32),
            pltpu.SemaphoreType.DMA,
            pltpu.SemaphoreType.DMA,
            pltpu.SemaphoreType.DMA,
            pltpu.SemaphoreType.DMA,
        ],
        compiler_params=pltpu.CompilerParams(use_tc_tiling_on_sc=False),
    )
    def agg(table_hbm, g_hbm, s_hbm, z_hbm, out_hbm,
            gi0, gi1, si0, si1, rows0_v, rows1_v, acc_sh,
            isem0, isem1, sem0, sem1):
        cid = lax.axis_index("c")
        sid = lax.axis_index("s")
        wid = sid * _NC + cid
        base = sid * rs
        for o, l in _chunks(rs, G):
            pltpu.sync_copy(z_hbm.at[pl.ds(0, l)],
                            acc_sh.at[pl.ds(base + o, l)])
        plsc.subcore_barrier()

        def ifetch(j, gbuf, sbuf, isem):
            a = pltpu.make_async_copy(g_hbm.at[wid, j], gbuf, isem)
            b = pltpu.make_async_copy(s_hbm.at[wid, j], sbuf, isem)
            return a, b

        def istart(j, gbuf, sbuf, isem):
            a, b = ifetch(j, gbuf, sbuf, isem)
            a.start()
            b.start()

        def iwait(j, gbuf, sbuf, isem):
            a, b = ifetch(j, gbuf, sbuf, isem)
            a.wait()
            b.wait()

        def gcopy(buf, ibuf, sem):
            return pltpu.make_async_copy(table_hbm.at[ibuf], buf, sem)

        istart(0, gi0, si0, isem0)
        iwait(0, gi0, si0, isem0)
        gcopy(rows0_v, gi0, sem0).start()
        istart(1, gi1, si1, isem1)

        def step(jj, carry):
            j0 = jj * 2
            j1 = j0 + 1
            gcopy(rows0_v, gi0, sem0).wait()
            iwait(j1, gi1, si1, isem1)
            gcopy(rows1_v, gi1, sem1).start()
            pltpu.sync_copy(rows0_v, acc_sh.at[si0], add=True)

            @pl.when(j0 + 2 < C)
            def _():
                istart(j0 + 2, gi0, si0, isem0)

            gcopy(rows1_v, gi1, sem1).wait()

            @pl.when(j0 + 2 < C)
            def _():
                iwait(j0 + 2, gi0, si0, isem0)
                gcopy(rows0_v, gi0, sem0).start()

            pltpu.sync_copy(rows1_v, acc_sh.at[si1], add=True)

            @pl.when(j1 + 2 < C)
            def _():
                istart(j1 + 2, gi1, si1, isem1)

            return carry

        lax.fori_loop(0, C // 2, step, 0)
        plsc.subcore_barrier()
        for o, l in _chunks(rs, G):
            pltpu.sync_copy(acc_sh.at[pl.ds(base + o, l)],
                            out_hbm.at[cid, pl.ds(base + o, l)])

    return agg


def _make_sc_segment_sum(S, G):
    rs = S // _NS
    C = _PER_W // G
    mesh = plsc.VectorSubcoreMesh(core_axis_name="c", subcore_axis_name="s")

    @functools.partial(
        pl.kernel,
        out_type=jax.ShapeDtypeStruct((_NC, S, _KE), jnp.float32),
        mesh=mesh,
        scratch_types=[
            pltpu.VMEM((C, G), jnp.int32),
            pltpu.VMEM((C, G), jnp.int32),
            pltpu.VMEM((G, _KE), jnp.float32),
            pltpu.VMEM((G, _KE), jnp.float32),
            pltpu.VMEM_SHARED((S, _KE), jnp.float32),
            pltpu.SemaphoreType.DMA,
            pltpu.SemaphoreType.DMA,
        ],
        compiler_params=pltpu.CompilerParams(use_tc_tiling_on_sc=False),
    )
    def agg(table_hbm, g_hbm, s_hbm, z_hbm, out_hbm,
            gi_v, si_v, rows0_v, rows1_v, acc_sh, sem0, sem1):
        cid = lax.axis_index("c")
        sid = lax.axis_index("s")
        wid = sid * _NC + cid
        pltpu.sync_copy(g_hbm.at[wid], gi_v)
        pltpu.sync_copy(s_hbm.at[wid], si_v)
        base = sid * rs
        for o, l in _chunks(rs, G):
            pltpu.sync_copy(z_hbm.at[pl.ds(0, l)],
                            acc_sh.at[pl.ds(base + o, l)])
        plsc.subcore_barrier()

        def gcopy(j, buf, sem):
            return pltpu.make_async_copy(table_hbm.at[gi_v.at[j]], buf, sem)

        gcopy(0, rows0_v, sem0).start()

        def step(jj, carry):
            j0 = jj * 2
            j1 = j0 + 1
            gcopy(j0, rows0_v, sem0).wait()
            gcopy(j1, rows1_v, sem1).start()
            pltpu.sync_copy(rows0_v, acc_sh.at[si_v.at[j0]], add=True)
            gcopy(j1, rows1_v, sem1).wait()

            @pl.when(j0 + 2 < C)
            def _():
                gcopy(j0 + 2, rows0_v, sem0).start()

            pltpu.sync_copy(rows1_v, acc_sh.at[si_v.at[j1]], add=True)
            return carry

        lax.fori_loop(0, C // 2, step, 0)
        plsc.subcore_barrier()
        for o, l in _chunks(rs, G):
            pltpu.sync_copy(acc_sh.at[pl.ds(base + o, l)],
                            out_hbm.at[cid, pl.ds(base + o, l)])

    return agg


def _combine_body(ep_ref, o_ref):
    v = ep_ref[...]
    s = v[0] + v[1]
    cnt = s[:, _K:_K + 1]
    scale = 1.0 / jnp.maximum(cnt, 1.0)
    lane = lax.broadcasted_iota(jnp.int32, s.shape, 1)
    feat = s * scale
    o_ref[...] = jnp.where(lane == _K, 1.0, jnp.where(lane > _K, 0.0, feat))


def _combine(e_part):
    S = e_part.shape[1]
    return pl.pallas_call(
        _combine_body,
        out_shape=jax.ShapeDtypeStruct((S, _KE), jnp.float32),
    )(e_part)


def _final_body(vp_ref, p_ref, w_ref, b_ref, o_ref):
    v = vp_ref[...]
    s = v[0] + v[1]
    cnt = s[:, _K:_K + 1]
    vfeat = s[:, :_K] / jnp.maximum(cnt, 1.0)
    g = jnp.dot(vfeat, w_ref[...], preferred_element_type=jnp.float32)
    g = g + b_ref[...]
    sp = jnp.maximum(p_ref[...][:, :_K], _EPS)
    ng = sp * g
    o_ref[...] = ng - jnp.mean(ng, axis=1, keepdims=True)


def _final(v_part, probs_ext, W, b2):
    br = 1000
    return pl.pallas_call(
        _final_body,
        grid=(_N // br,),
        in_specs=[
            pl.BlockSpec((_NC, br, _KE), lambda i: (0, i, 0)),
            pl.BlockSpec((br, _KE), lambda i: (i, 0)),
            pl.BlockSpec((_K, _K), lambda i: (0, 0)),
            pl.BlockSpec((1, _K), lambda i: (0, 0)),
        ],
        out_specs=pl.BlockSpec((br, _K), lambda i: (i, 0)),
        out_shape=jax.ShapeDtypeStruct((_N, _K), jnp.float32),
    )(v_part, probs_ext, W, b2)


_GE = 125
_GN = 125
_sc_edge_agg = _make_sc_segment_sum(_MP, _GE)
_sc_node_agg = _make_sc_segment_sum_stream_idx(_NP, _GN)


def kernel(t, y, incidence, W, b):
    del t
    node_e = incidence[0].reshape(_NW, _PER_W // _GE, _GE)
    edge_e = incidence[1].reshape(_NW, _PER_W // _GE, _GE)
    node_n = incidence[0].reshape(_NW, _PER_W // _GN, _GN)
    edge_n = incidence[1].reshape(_NW, _PER_W // _GN, _GN)
    zrows_e = jnp.zeros((_GE, _KE), jnp.float32)
    zrows_n = jnp.zeros((_GN, _KE), jnp.float32)
    probs_ext = _softmax_ext(y)
    e_part = _sc_edge_agg(probs_ext, node_e, edge_e, zrows_e)
    e_feat = _combine(e_part)
    v_part = _sc_node_agg(e_feat, edge_n, node_n, zrows_n)
    return _final(v_part, probs_ext, W, b.reshape(1, _K))


# speedup vs baseline: 1.0237x; 1.0237x over previous
import functools

import jax
import jax.numpy as jnp
from jax import lax
from jax.experimental import pallas as pl
from jax.experimental.pallas import tpu as pltpu
from jax.experimental.pallas import tpu_sc as plsc

_N = 10000
_M = 5000
_NNZ = 320000
_K = 128
_KE = 144
_MP = 5120
_NP = 10240
_NC = 2
_NS = 16
_NW = _NC * _NS
_PER_W = _NNZ // _NW
_EPS = 1e-12


def _chunks(total, step):
    out, o = [], 0
    while o < total:
        l = min(step, total - o)
        out.append((o, l))
        o += l
    return out


def _softmax_ext_body(y_ref, o_ref):
    y = y_ref[...]
    m = jnp.max(y, axis=1, keepdims=True)
    e = jnp.exp(y - m)
    p = e / jnp.sum(e, axis=1, keepdims=True)
    lane = lax.broadcasted_iota(jnp.int32, (y.shape[0], _KE - _K), 1)
    ext = jnp.where(lane == 0, 1.0, 0.0).astype(jnp.float32)
    o_ref[...] = jnp.concatenate([p, ext], axis=1)


def _softmax_ext(y):
    br = 1000
    return pl.pallas_call(
        _softmax_ext_body,
        grid=(_N // br,),
        in_specs=[pl.BlockSpec((br, _K), lambda i: (i, 0))],
        out_specs=pl.BlockSpec((br, _KE), lambda i: (i, 0)),
        out_shape=jax.ShapeDtypeStruct((_N, _KE), jnp.float32),
    )(y)


def _make_sc_segment_sum_stream_idx(S, G):
    rs = S // _NS
    C = _PER_W // G
    mesh = plsc.VectorSubcoreMesh(core_axis_name="c", subcore_axis_name="s")

    @functools.partial(
        pl.kernel,
        out_type=jax.ShapeDtypeStruct((_NC, S, _KE), jnp.float32),
        mesh=mesh,
        scratch_types=[
            pltpu.VMEM((G,), jnp.int32),
            pltpu.VMEM((G,), jnp.int32),
            pltpu.VMEM((G,), jnp.int32),
            pltpu.VMEM((G,), jnp.int32),
            pltpu.VMEM((G, _KE), jnp.float32),
            pltpu.VMEM((G, _KE), jnp.float32),
            pltpu.VMEM_SHARED((S, _KE), jnp.float32),
            pltpu.SemaphoreType.DMA,
            pltpu.SemaphoreType.DMA,
            pltpu.SemaphoreType.DMA,
            pltpu.SemaphoreType.DMA,
        ],
        compiler_params=pltpu.CompilerParams(use_tc_tiling_on_sc=False),
    )
    def agg(table_hbm, g_hbm, s_hbm, z_hbm, out_hbm,
            gi0, gi1, si0, si1, rows0_v, rows1_v, acc_sh,
            isem0, isem1, sem0, sem1):
        cid = lax.axis_index("c")
        sid = lax.axis_index("s")
        wid = sid * _NC + cid
        pltpu.sync_copy(z_hbm, rows0_v)
        base = sid * rs
        for o, l in _chunks(rs, min(G, 120) // 8 * 8):
            pltpu.sync_copy(rows0_v.at[pl.ds(0, l)],
                            acc_sh.at[pl.ds(base + o, l)])
        plsc.subcore_barrier()

        def ifetch(j, gbuf, sbuf, isem):
            a = pltpu.make_async_copy(g_hbm.at[wid, j], gbuf, isem)
            b = pltpu.make_async_copy(s_hbm.at[wid, j], sbuf, isem)
            return a, b

        def istart(j, gbuf, sbuf, isem):
            a, b = ifetch(j, gbuf, sbuf, isem)
            a.start()
            b.start()

        def iwait(j, gbuf, sbuf, isem):
            a, b = ifetch(j, gbuf, sbuf, isem)
            a.wait()
            b.wait()

        def gcopy(buf, ibuf, sem):
            return pltpu.make_async_copy(table_hbm.at[ibuf], buf, sem)

        istart(0, gi0, si0, isem0)
        iwait(0, gi0, si0, isem0)
        gcopy(rows0_v, gi0, sem0).start()
        istart(1, gi1, si1, isem1)

        def step(jj, carry):
            j0 = jj * 2
            j1 = j0 + 1
            gcopy(rows0_v, gi0, sem0).wait()
            iwait(j1, gi1, si1, isem1)
            gcopy(rows1_v, gi1, sem1).start()
            pltpu.sync_copy(rows0_v, acc_sh.at[si0], add=True)

            @pl.when(j0 + 2 < C)
            def _():
                istart(j0 + 2, gi0, si0, isem0)

            gcopy(rows1_v, gi1, sem1).wait()

            @pl.when(j0 + 2 < C)
            def _():
                iwait(j0 + 2, gi0, si0, isem0)
                gcopy(rows0_v, gi0, sem0).start()

            pltpu.sync_copy(rows1_v, acc_sh.at[si1], add=True)

            @pl.when(j1 + 2 < C)
            def _():
                istart(j1 + 2, gi1, si1, isem1)

            return carry

        lax.fori_loop(0, C // 2, step, 0)
        plsc.subcore_barrier()
        for o, l in _chunks(rs, min(G, 120) // 8 * 8):
            pltpu.sync_copy(acc_sh.at[pl.ds(base + o, l)],
                            rows0_v.at[pl.ds(0, l)])
            pltpu.sync_copy(rows0_v.at[pl.ds(0, l)],
                            out_hbm.at[cid, pl.ds(base + o, l)])

    return agg


def _make_sc_segment_sum(S, G):
    rs = S // _NS
    C = _PER_W // G
    mesh = plsc.VectorSubcoreMesh(core_axis_name="c", subcore_axis_name="s")

    @functools.partial(
        pl.kernel,
        out_type=jax.ShapeDtypeStruct((_NC, S, _KE), jnp.float32),
        mesh=mesh,
        scratch_types=[
            pltpu.VMEM((C, G), jnp.int32),
            pltpu.VMEM((C, G), jnp.int32),
            pltpu.VMEM((G, _KE), jnp.float32),
            pltpu.VMEM((G, _KE), jnp.float32),
            pltpu.VMEM_SHARED((S, _KE), jnp.float32),
            pltpu.SemaphoreType.DMA,
            pltpu.SemaphoreType.DMA,
        ],
        compiler_params=pltpu.CompilerParams(use_tc_tiling_on_sc=False),
    )
    def agg(table_hbm, g_hbm, s_hbm, z_hbm, out_hbm,
            gi_v, si_v, rows0_v, rows1_v, acc_sh, sem0, sem1):
        cid = lax.axis_index("c")
        sid = lax.axis_index("s")
        wid = sid * _NC + cid
        pltpu.sync_copy(g_hbm.at[wid], gi_v)
        pltpu.sync_copy(s_hbm.at[wid], si_v)
        pltpu.sync_copy(z_hbm, rows0_v)
        base = sid * rs
        for o, l in _chunks(rs, min(G, 120) // 8 * 8):
            pltpu.sync_copy(rows0_v.at[pl.ds(0, l)],
                            acc_sh.at[pl.ds(base + o, l)])
        plsc.subcore_barrier()

        def gcopy(j, buf, sem):
            return pltpu.make_async_copy(table_hbm.at[gi_v.at[j]], buf, sem)

        gcopy(0, rows0_v, sem0).start()

        def step(jj, carry):
            j0 = jj * 2
            j1 = j0 + 1
            gcopy(j0, rows0_v, sem0).wait()
            gcopy(j1, rows1_v, sem1).start()
            pltpu.sync_copy(rows0_v, acc_sh.at[si_v.at[j0]], add=True)
            gcopy(j1, rows1_v, sem1).wait()

            @pl.when(j0 + 2 < C)
            def _():
                gcopy(j0 + 2, rows0_v, sem0).start()

            pltpu.sync_copy(rows1_v, acc_sh.at[si_v.at[j1]], add=True)
            return carry

        lax.fori_loop(0, C // 2, step, 0)
        plsc.subcore_barrier()
        for o, l in _chunks(rs, min(G, 120) // 8 * 8):
            pltpu.sync_copy(acc_sh.at[pl.ds(base + o, l)],
                            rows0_v.at[pl.ds(0, l)])
            pltpu.sync_copy(rows0_v.at[pl.ds(0, l)],
                            out_hbm.at[cid, pl.ds(base + o, l)])

    return agg


def _combine_body(ep_ref, o_ref):
    v = ep_ref[...]
    s = v[0] + v[1]
    cnt = s[:, _K:_K + 1]
    scale = 1.0 / jnp.maximum(cnt, 1.0)
    lane = lax.broadcasted_iota(jnp.int32, s.shape, 1)
    feat = s * scale
    o_ref[...] = jnp.where(lane == _K, 1.0, jnp.where(lane > _K, 0.0, feat))


def _combine(e_part):
    S = e_part.shape[1]
    return pl.pallas_call(
        _combine_body,
        out_shape=jax.ShapeDtypeStruct((S, _KE), jnp.float32),
    )(e_part)


def _final_body(vp_ref, p_ref, w_ref, b_ref, o_ref):
    v = vp_ref[...]
    s = v[0] + v[1]
    cnt = s[:, _K:_K + 1]
    vfeat = s[:, :_K] / jnp.maximum(cnt, 1.0)
    g = jnp.dot(vfeat, w_ref[...], preferred_element_type=jnp.float32)
    g = g + b_ref[...]
    sp = jnp.maximum(p_ref[...][:, :_K], _EPS)
    ng = sp * g
    o_ref[...] = ng - jnp.mean(ng, axis=1, keepdims=True)


def _final(v_part, probs_ext, W, b2):
    br = 1000
    return pl.pallas_call(
        _final_body,
        grid=(_N // br,),
        in_specs=[
            pl.BlockSpec((_NC, br, _KE), lambda i: (0, i, 0)),
            pl.BlockSpec((br, _KE), lambda i: (i, 0)),
            pl.BlockSpec((_K, _K), lambda i: (0, 0)),
            pl.BlockSpec((1, _K), lambda i: (0, 0)),
        ],
        out_specs=pl.BlockSpec((br, _K), lambda i: (i, 0)),
        out_shape=jax.ShapeDtypeStruct((_N, _K), jnp.float32),
    )(v_part, probs_ext, W, b2)


_GE = 125
_GN = 125
_sc_edge_agg = _make_sc_segment_sum(_MP, _GE)
_sc_node_agg = _make_sc_segment_sum_stream_idx(_NP, _GN)


def kernel(t, y, incidence, W, b):
    del t
    node_e = incidence[0].reshape(_NW, _PER_W // _GE, _GE)
    edge_e = incidence[1].reshape(_NW, _PER_W // _GE, _GE)
    node_n = incidence[0].reshape(_NW, _PER_W // _GN, _GN)
    edge_n = incidence[1].reshape(_NW, _PER_W // _GN, _GN)
    zrows_e = jnp.zeros((_GE, _KE), jnp.float32)
    zrows_n = jnp.zeros((_GN, _KE), jnp.float32)
    probs_ext = _softmax_ext(y)
    e_part = _sc_edge_agg(probs_ext, node_e, edge_e, zrows_e)
    e_feat = _combine(e_part)
    v_part = _sc_node_agg(e_feat, edge_n, node_n, zrows_n)
    return _final(v_part, probs_ext, W, b.reshape(1, _K))
